# baseline (device time: 25878 ns/iter reference)
import jax
import jax.numpy as jnp
from jax import lax
from jax.experimental import pallas as pl
from jax.experimental.pallas import tpu as pltpu

N_DEV = 4
B, SQ, SKV, HQ_LOCAL, DH = 2, 256, 256, 4, 64
DMODEL = 512
WINDOW = 128
SCALE = 0.125


def kernel(x, Wq, K_ext, V_ext, Wo):
    K2 = K_ext.reshape(B, SKV, 16 * DH)
    V2 = V_ext.reshape(B, SKV, 16 * DH)

    def body(x_ref, wq_ref, k_ref, v_ref, wo_ref, out_ref,
             ctx_ref, scat_ref, send_a, recv_a, send_b, recv_b):
        me = lax.axis_index("i")

        barrier_sem = pltpu.get_barrier_semaphore()
        for d in range(1, N_DEV):
            pl.semaphore_signal(
                barrier_sem, inc=1,
                device_id=((me + d) % N_DEV,),
                device_id_type=pl.DeviceIdType.MESH,
            )
        pl.semaphore_wait(barrier_sem, N_DEV - 1)

        QR = SQ // N_DEV

        qi = lax.broadcasted_iota(jnp.int32, (SQ, SKV), 0)
        ki = lax.broadcasted_iota(jnp.int32, (SQ, SKV), 1)
        mask = jnp.abs(qi - ki) <= WINDOW

        wq = wq_ref[:, :]
        wo = wo_ref[:, :]
        for b in range(B):
            q_b = jnp.dot(x_ref[b], wq, preferred_element_type=jnp.float32)
            my_lanes = pl.ds(me * HQ_LOCAL * DH, HQ_LOCAL * DH)
            k_my = k_ref[b, :, my_lanes]
            v_my = v_ref[b, :, my_lanes]
            for h in range(HQ_LOCAL):
                q_bh = q_b[:, h * DH:(h + 1) * DH]
                k_bh = k_my[:, h * DH:(h + 1) * DH]
                s = lax.dot_general(
                    q_bh, k_bh, (((1,), (1,)), ((), ())),
                    preferred_element_type=jnp.float32,
                ) * SCALE
                w = jnp.where(mask, jnp.exp(s), 0.0)
                w = w / jnp.sum(w, axis=1, keepdims=True)
                ctx_ref[b, :, h * DH:(h + 1) * DH] = jnp.dot(
                    w, v_my[:, h * DH:(h + 1) * DH],
                    preferred_element_type=jnp.float32,
                )

        for b in range(B):
            out_ref[b] = jnp.dot(ctx_ref[b], wo, preferred_element_type=jnp.float32)

        sends_a = []
        for d in range(1, N_DEV):
            t = (me + d) % N_DEV
            rows = pl.ds(t * QR, QR)
            rdma = pltpu.make_async_remote_copy(
                src_ref=out_ref.at[:, rows, :],
                dst_ref=scat_ref.at[N_DEV - 1 - d],
                send_sem=send_a.at[d - 1],
                recv_sem=recv_a.at[N_DEV - 1 - d],
                device_id=(t,),
                device_id_type=pl.DeviceIdType.MESH,
            )
            rdma.start()
            sends_a.append(rdma)

        for r in range(N_DEV - 1):
            pltpu.make_async_remote_copy(
                src_ref=scat_ref.at[r], dst_ref=scat_ref.at[r],
                send_sem=send_a.at[r], recv_sem=recv_a.at[r],
                device_id=(me,), device_id_type=pl.DeviceIdType.MESH,
            ).wait_recv()
        for rdma in sends_a:
            rdma.wait_send()

        my_rows = pl.ds(me * QR, QR)
        out_ref[:, my_rows, :] = (
            out_ref[:, my_rows, :] + scat_ref[0] + scat_ref[1] + scat_ref[2]
        )

        sends_b = []
        for d in range(1, N_DEV):
            t = (me + d) % N_DEV
            rdma = pltpu.make_async_remote_copy(
                src_ref=out_ref.at[:, my_rows, :],
                dst_ref=out_ref.at[:, my_rows, :],
                send_sem=send_b.at[d - 1],
                recv_sem=recv_b.at[N_DEV - 1 - d],
                device_id=(t,),
                device_id_type=pl.DeviceIdType.MESH,
            )
            rdma.start()
            sends_b.append(rdma)
        for r in range(N_DEV - 1):
            s = (me + 1 + r) % N_DEV
            pltpu.make_async_remote_copy(
                src_ref=out_ref.at[:, pl.ds(s * QR, QR), :],
                dst_ref=out_ref.at[:, pl.ds(s * QR, QR), :],
                send_sem=send_b.at[r], recv_sem=recv_b.at[r],
                device_id=(me,), device_id_type=pl.DeviceIdType.MESH,
            ).wait_recv()
        for rdma in sends_b:
            rdma.wait_send()

    return pl.pallas_call(
        body,
        out_shape=jax.ShapeDtypeStruct((B, SQ, DMODEL), jnp.float32),
        in_specs=[pl.BlockSpec(memory_space=pltpu.VMEM)] * 5,
        out_specs=pl.BlockSpec(memory_space=pltpu.VMEM),
        scratch_shapes=[
            pltpu.VMEM((B, SQ, HQ_LOCAL * DH), jnp.float32),
            pltpu.VMEM((N_DEV - 1, B, SQ // N_DEV, DMODEL), jnp.float32),
            pltpu.SemaphoreType.DMA((N_DEV - 1,)),
            pltpu.SemaphoreType.DMA((N_DEV - 1,)),
            pltpu.SemaphoreType.DMA((N_DEV - 1,)),
            pltpu.SemaphoreType.DMA((N_DEV - 1,)),
        ],
        compiler_params=pltpu.CompilerParams(collective_id=0),
    )(x, Wq, K2, V2, Wo)


# device time: 22163 ns/iter; 1.1676x vs baseline; 1.1676x over previous
import jax
import jax.numpy as jnp
from jax import lax
from jax.experimental import pallas as pl
from jax.experimental.pallas import tpu as pltpu

N_DEV = 4
B, SQ, SKV, HQ_LOCAL, DH = 2, 256, 256, 4, 64
DMODEL = 512
WINDOW = 128
SCALE = 0.125


def kernel(x, Wq, K_ext, V_ext, Wo):
    my = lax.axis_index("i")
    K_sh = lax.dynamic_slice_in_dim(K_ext, my * HQ_LOCAL, HQ_LOCAL, axis=2)
    V_sh = lax.dynamic_slice_in_dim(V_ext, my * HQ_LOCAL, HQ_LOCAL, axis=2)
    K_sh = jnp.transpose(K_sh, (0, 2, 1, 3))
    V_sh = jnp.transpose(V_sh, (0, 2, 1, 3))

    def body(x_ref, wq_ref, k_ref, v_ref, wo_ref, out_ref,
             ctx_ref, scat_ref, send_a, recv_a, send_b, recv_b):
        me = lax.axis_index("i")

        barrier_sem = pltpu.get_barrier_semaphore()
        for d in range(1, N_DEV):
            pl.semaphore_signal(
                barrier_sem, inc=1,
                device_id=((me + d) % N_DEV,),
                device_id_type=pl.DeviceIdType.MESH,
            )
        pl.semaphore_wait(barrier_sem, N_DEV - 1)

        QR = SQ // N_DEV

        qi = lax.broadcasted_iota(jnp.int32, (SQ, SKV), 0)
        ki = lax.broadcasted_iota(jnp.int32, (SQ, SKV), 1)
        mask = jnp.abs(qi - ki) <= WINDOW

        wq = wq_ref[:, :]
        wo = wo_ref[:, :]
        for b in range(B):
            q_b = jnp.dot(x_ref[b], wq, preferred_element_type=jnp.float32)
            for h in range(HQ_LOCAL):
                q_bh = q_b[:, h * DH:(h + 1) * DH]
                k_bh = k_ref[b, h]
                s = lax.dot_general(
                    q_bh, k_bh, (((1,), (1,)), ((), ())),
                    preferred_element_type=jnp.float32,
                ) * SCALE
                w = jnp.where(mask, jnp.exp(s), 0.0)
                w = w / jnp.sum(w, axis=1, keepdims=True)
                ctx_ref[b, :, h * DH:(h + 1) * DH] = jnp.dot(
                    w, v_ref[b, h], preferred_element_type=jnp.float32
                )

        for b in range(B):
            out_ref[b] = jnp.dot(ctx_ref[b], wo, preferred_element_type=jnp.float32)

        CH = 2
        CW = DMODEL // CH
        NP = N_DEV - 1
        my_rows = pl.ds(me * QR, QR)

        sends = []
        for c in range(CH):
            cols = pl.ds(c * CW, CW)
            for d in range(1, N_DEV):
                t = (me + d) % N_DEV
                rdma = pltpu.make_async_remote_copy(
                    src_ref=out_ref.at[:, pl.ds(t * QR, QR), cols],
                    dst_ref=scat_ref.at[N_DEV - 1 - d, :, :, cols],
                    send_sem=send_a.at[c * NP + d - 1],
                    recv_sem=recv_a.at[c * NP + N_DEV - 1 - d],
                    device_id=(t,),
                    device_id_type=pl.DeviceIdType.MESH,
                )
                rdma.start()
                sends.append(rdma)

        for c in range(CH):
            cols = pl.ds(c * CW, CW)
            for r in range(NP):
                pltpu.make_async_remote_copy(
                    src_ref=scat_ref.at[r, :, :, cols],
                    dst_ref=scat_ref.at[r, :, :, cols],
                    send_sem=send_a.at[c * NP + r],
                    recv_sem=recv_a.at[c * NP + r],
                    device_id=(me,), device_id_type=pl.DeviceIdType.MESH,
                ).wait_recv()
            out_ref[:, my_rows, cols] = (
                out_ref[:, my_rows, cols]
                + scat_ref[0, :, :, cols]
                + scat_ref[1, :, :, cols]
                + scat_ref[2, :, :, cols]
            )
            for d in range(1, N_DEV):
                t = (me + d) % N_DEV
                rdma = pltpu.make_async_remote_copy(
                    src_ref=out_ref.at[:, my_rows, cols],
                    dst_ref=out_ref.at[:, my_rows, cols],
                    send_sem=send_b.at[c * NP + d - 1],
                    recv_sem=recv_b.at[c * NP + N_DEV - 1 - d],
                    device_id=(t,),
                    device_id_type=pl.DeviceIdType.MESH,
                )
                rdma.start()
                sends.append(rdma)

        for c in range(CH):
            cols = pl.ds(c * CW, CW)
            for r in range(NP):
                s = (me + 1 + r) % N_DEV
                pltpu.make_async_remote_copy(
                    src_ref=out_ref.at[:, pl.ds(s * QR, QR), cols],
                    dst_ref=out_ref.at[:, pl.ds(s * QR, QR), cols],
                    send_sem=send_b.at[c * NP + r],
                    recv_sem=recv_b.at[c * NP + r],
                    device_id=(me,), device_id_type=pl.DeviceIdType.MESH,
                ).wait_recv()
        for rdma in sends:
            rdma.wait_send()

    return pl.pallas_call(
        body,
        out_shape=jax.ShapeDtypeStruct((B, SQ, DMODEL), jnp.float32),
        in_specs=[pl.BlockSpec(memory_space=pltpu.VMEM)] * 5,
        out_specs=pl.BlockSpec(memory_space=pltpu.VMEM),
        scratch_shapes=[
            pltpu.VMEM((B, SQ, HQ_LOCAL * DH), jnp.float32),
            pltpu.VMEM((N_DEV - 1, B, SQ // N_DEV, DMODEL), jnp.float32),
            pltpu.SemaphoreType.DMA((2 * (N_DEV - 1),)),
            pltpu.SemaphoreType.DMA((2 * (N_DEV - 1),)),
            pltpu.SemaphoreType.DMA((2 * (N_DEV - 1),)),
            pltpu.SemaphoreType.DMA((2 * (N_DEV - 1),)),
        ],
        compiler_params=pltpu.CompilerParams(collective_id=0),
    )(x, Wq, K_sh, V_sh, Wo)


# device time: 21392 ns/iter; 1.2097x vs baseline; 1.0360x over previous
import jax
import jax.numpy as jnp
from jax import lax
from jax.experimental import pallas as pl
from jax.experimental.pallas import tpu as pltpu

N_DEV = 4
B, SQ, SKV, HQ_LOCAL, DH = 2, 256, 256, 4, 64
DMODEL = 512
WINDOW = 128
SCALE = 0.125
CH = 4


def kernel(x, Wq, K_ext, V_ext, Wo):
    my = lax.axis_index("i")
    K_sh = lax.dynamic_slice_in_dim(K_ext, my * HQ_LOCAL, HQ_LOCAL, axis=2)
    V_sh = lax.dynamic_slice_in_dim(V_ext, my * HQ_LOCAL, HQ_LOCAL, axis=2)
    K_sh = jnp.transpose(K_sh, (0, 2, 1, 3))
    V_sh = jnp.transpose(V_sh, (0, 2, 1, 3))

    def body(x_ref, wq_ref, k_ref, v_ref, wo_ref, out_ref,
             ctx_ref, scat_ref, send_a, recv_a, send_b, recv_b):
        me = lax.axis_index("i")

        barrier_sem = pltpu.get_barrier_semaphore()
        for d in range(1, N_DEV):
            pl.semaphore_signal(
                barrier_sem, inc=1,
                device_id=((me + d) % N_DEV,),
                device_id_type=pl.DeviceIdType.MESH,
            )

        QR = SQ // N_DEV

        qi = lax.broadcasted_iota(jnp.int32, (SQ, SKV), 0)
        ki = lax.broadcasted_iota(jnp.int32, (SQ, SKV), 1)
        mask = jnp.abs(qi - ki) <= WINDOW

        wq = wq_ref[:, :]
        wo = wo_ref[:, :]
        for b in range(B):
            q_b = jnp.dot(x_ref[b], wq, preferred_element_type=jnp.float32)
            for h in range(HQ_LOCAL):
                q_bh = q_b[:, h * DH:(h + 1) * DH]
                k_bh = k_ref[b, h]
                s = lax.dot_general(
                    q_bh, k_bh, (((1,), (1,)), ((), ())),
                    preferred_element_type=jnp.float32,
                ) * SCALE
                w = jnp.where(mask, jnp.exp(s), 0.0)
                w = w / jnp.sum(w, axis=1, keepdims=True)
                ctx_ref[b, :, h * DH:(h + 1) * DH] = jnp.dot(
                    w, v_ref[b, h], preferred_element_type=jnp.float32
                )

        for b in range(B):
            out_ref[b] = jnp.dot(ctx_ref[b], wo, preferred_element_type=jnp.float32)

        CW = DMODEL // CH
        NP = N_DEV - 1
        my_rows = pl.ds(me * QR, QR)

        pl.semaphore_wait(barrier_sem, N_DEV - 1)

        sends = []
        for c in range(CH):
            cols = pl.ds(c * CW, CW)
            for d in range(1, N_DEV):
                t = (me + d) % N_DEV
                rdma = pltpu.make_async_remote_copy(
                    src_ref=out_ref.at[:, pl.ds(t * QR, QR), cols],
                    dst_ref=scat_ref.at[N_DEV - 1 - d, :, :, cols],
                    send_sem=send_a.at[c * NP + d - 1],
                    recv_sem=recv_a.at[c * NP + N_DEV - 1 - d],
                    device_id=(t,),
                    device_id_type=pl.DeviceIdType.MESH,
                )
                rdma.start()
                sends.append(rdma)

        for c in range(CH):
            cols = pl.ds(c * CW, CW)
            for r in range(NP):
                pltpu.make_async_remote_copy(
                    src_ref=scat_ref.at[r, :, :, cols],
                    dst_ref=scat_ref.at[r, :, :, cols],
                    send_sem=send_a.at[c * NP + r],
                    recv_sem=recv_a.at[c * NP + r],
                    device_id=(me,), device_id_type=pl.DeviceIdType.MESH,
                ).wait_recv()
            out_ref[:, my_rows, cols] = (
                out_ref[:, my_rows, cols]
                + scat_ref[0, :, :, cols]
                + scat_ref[1, :, :, cols]
                + scat_ref[2, :, :, cols]
            )
            for d in range(1, N_DEV):
                t = (me + d) % N_DEV
                rdma = pltpu.make_async_remote_copy(
                    src_ref=out_ref.at[:, my_rows, cols],
                    dst_ref=out_ref.at[:, my_rows, cols],
                    send_sem=send_b.at[c * NP + d - 1],
                    recv_sem=recv_b.at[c * NP + N_DEV - 1 - d],
                    device_id=(t,),
                    device_id_type=pl.DeviceIdType.MESH,
                )
                rdma.start()
                sends.append(rdma)

        for c in range(CH):
            cols = pl.ds(c * CW, CW)
            for r in range(NP):
                s = (me + 1 + r) % N_DEV
                pltpu.make_async_remote_copy(
                    src_ref=out_ref.at[:, pl.ds(s * QR, QR), cols],
                    dst_ref=out_ref.at[:, pl.ds(s * QR, QR), cols],
                    send_sem=send_b.at[c * NP + r],
                    recv_sem=recv_b.at[c * NP + r],
                    device_id=(me,), device_id_type=pl.DeviceIdType.MESH,
                ).wait_recv()
        for rdma in sends:
            rdma.wait_send()

    return pl.pallas_call(
        body,
        out_shape=jax.ShapeDtypeStruct((B, SQ, DMODEL), jnp.float32),
        in_specs=[pl.BlockSpec(memory_space=pltpu.VMEM)] * 5,
        out_specs=pl.BlockSpec(memory_space=pltpu.VMEM),
        scratch_shapes=[
            pltpu.VMEM((B, SQ, HQ_LOCAL * DH), jnp.float32),
            pltpu.VMEM((N_DEV - 1, B, SQ // N_DEV, DMODEL), jnp.float32),
            pltpu.SemaphoreType.DMA((CH * (N_DEV - 1),)),
            pltpu.SemaphoreType.DMA((CH * (N_DEV - 1),)),
            pltpu.SemaphoreType.DMA((CH * (N_DEV - 1),)),
            pltpu.SemaphoreType.DMA((CH * (N_DEV - 1),)),
        ],
        compiler_params=pltpu.CompilerParams(collective_id=0),
    )(x, Wq, K_sh, V_sh, Wo)


# device time: 17491 ns/iter; 1.4795x vs baseline; 1.2230x over previous
import jax
import jax.numpy as jnp
from jax import lax
from jax.experimental import pallas as pl
from jax.experimental.pallas import tpu as pltpu

N_DEV = 4
B, SQ, SKV, HQ_LOCAL, DH = 2, 256, 256, 4, 64
DMODEL = 512
WINDOW = 128
SCALE = 0.125
CH = 4
QR = SQ // N_DEV
CW = DMODEL // CH
NP = N_DEV - 1


def kernel(x, Wq, K_ext, V_ext, Wo):
    my = lax.axis_index("i")
    K_sh = lax.dynamic_slice_in_dim(K_ext, my * HQ_LOCAL, HQ_LOCAL, axis=2)
    V_sh = lax.dynamic_slice_in_dim(V_ext, my * HQ_LOCAL, HQ_LOCAL, axis=2)
    K_sh = jnp.transpose(K_sh, (0, 2, 1, 3)).astype(jnp.bfloat16)
    V_sh = jnp.transpose(V_sh, (0, 2, 1, 3)).astype(jnp.bfloat16)
    x16 = x.astype(jnp.bfloat16)
    Wq16 = Wq.astype(jnp.bfloat16)
    Wo16 = Wo.astype(jnp.bfloat16)

    def body(x_ref, wq_ref, k_ref, v_ref, wo_ref, out_ref,
             ctx_ref, part_ref, fin_ref, scat_ref,
             send_a, recv_a, send_b, recv_b):
        me = lax.axis_index("i")

        barrier_sem = pltpu.get_barrier_semaphore()
        for d in range(1, N_DEV):
            pl.semaphore_signal(
                barrier_sem, inc=1,
                device_id=((me + d) % N_DEV,),
                device_id_type=pl.DeviceIdType.MESH,
            )

        qi = lax.broadcasted_iota(jnp.int32, (SQ, SKV), 0)
        ki = lax.broadcasted_iota(jnp.int32, (SQ, SKV), 1)
        mask = jnp.abs(qi - ki) <= WINDOW

        wq = wq_ref[:, :]
        wo = wo_ref[:, :]
        for b in range(B):
            q_b = jnp.dot(x_ref[b], wq,
                          preferred_element_type=jnp.float32)
            q_b16 = q_b.astype(jnp.bfloat16)
            for h in range(HQ_LOCAL):
                q_bh = q_b16[:, h * DH:(h + 1) * DH]
                k_bh = k_ref[b, h]
                s = lax.dot_general(
                    q_bh, k_bh, (((1,), (1,)), ((), ())),
                    preferred_element_type=jnp.float32,
                ) * SCALE
                w = jnp.where(mask, jnp.exp(s), 0.0)
                w = w / jnp.sum(w, axis=1, keepdims=True)
                ctx_ref[b, :, h * DH:(h + 1) * DH] = jnp.dot(
                    w.astype(jnp.bfloat16), v_ref[b, h],
                    preferred_element_type=jnp.float32,
                ).astype(jnp.bfloat16)
        for b in range(B):
            p_b = jnp.dot(ctx_ref[b], wo, preferred_element_type=jnp.float32)
            out_ref[b] = p_b
            part_ref[b] = p_b.astype(jnp.bfloat16)

        my_rows = pl.ds(me * QR, QR)

        pl.semaphore_wait(barrier_sem, N_DEV - 1)

        sends = []
        for c in range(CH):
            cols = pl.ds(c * CW, CW)
            for d in range(1, N_DEV):
                t = (me + d) % N_DEV
                rdma = pltpu.make_async_remote_copy(
                    src_ref=part_ref.at[:, pl.ds(t * QR, QR), cols],
                    dst_ref=scat_ref.at[N_DEV - 1 - d, :, :, cols],
                    send_sem=send_a.at[c * NP + d - 1],
                    recv_sem=recv_a.at[c * NP + N_DEV - 1 - d],
                    device_id=(t,),
                    device_id_type=pl.DeviceIdType.MESH,
                )
                rdma.start()
                sends.append(rdma)

        for c in range(CH):
            cols = pl.ds(c * CW, CW)
            for r in range(NP):
                pltpu.make_async_remote_copy(
                    src_ref=scat_ref.at[r, :, :, cols],
                    dst_ref=scat_ref.at[r, :, :, cols],
                    send_sem=send_a.at[c * NP + r],
                    recv_sem=recv_a.at[c * NP + r],
                    device_id=(me,), device_id_type=pl.DeviceIdType.MESH,
                ).wait_recv()
            red = (
                out_ref[:, my_rows, cols]
                + scat_ref[0, :, :, cols].astype(jnp.float32)
                + scat_ref[1, :, :, cols].astype(jnp.float32)
                + scat_ref[2, :, :, cols].astype(jnp.float32)
            )
            out_ref[:, my_rows, cols] = red
            fin_ref[:, my_rows, cols] = red.astype(jnp.bfloat16)
            for d in range(1, N_DEV):
                t = (me + d) % N_DEV
                rdma = pltpu.make_async_remote_copy(
                    src_ref=fin_ref.at[:, my_rows, cols],
                    dst_ref=fin_ref.at[:, my_rows, cols],
                    send_sem=send_b.at[c * NP + d - 1],
                    recv_sem=recv_b.at[c * NP + N_DEV - 1 - d],
                    device_id=(t,),
                    device_id_type=pl.DeviceIdType.MESH,
                )
                rdma.start()
                sends.append(rdma)

        for c in range(CH):
            cols = pl.ds(c * CW, CW)
            for r in range(NP):
                s = (me + 1 + r) % N_DEV
                pltpu.make_async_remote_copy(
                    src_ref=fin_ref.at[:, pl.ds(s * QR, QR), cols],
                    dst_ref=fin_ref.at[:, pl.ds(s * QR, QR), cols],
                    send_sem=send_b.at[c * NP + r],
                    recv_sem=recv_b.at[c * NP + r],
                    device_id=(me,), device_id_type=pl.DeviceIdType.MESH,
                ).wait_recv()
        for r in range(NP):
            s_rows = pl.ds(((me + 1 + r) % N_DEV) * QR, QR)
            out_ref[:, s_rows, :] = fin_ref[:, s_rows, :].astype(jnp.float32)
        for rdma in sends:
            rdma.wait_send()

    return pl.pallas_call(
        body,
        out_shape=jax.ShapeDtypeStruct((B, SQ, DMODEL), jnp.float32),
        in_specs=[pl.BlockSpec(memory_space=pltpu.VMEM)] * 5,
        out_specs=pl.BlockSpec(memory_space=pltpu.VMEM),
        scratch_shapes=[
            pltpu.VMEM((B, SQ, HQ_LOCAL * DH), jnp.bfloat16),
            pltpu.VMEM((B, SQ, DMODEL), jnp.bfloat16),
            pltpu.VMEM((B, SQ, DMODEL), jnp.bfloat16),
            pltpu.VMEM((NP, B, QR, DMODEL), jnp.bfloat16),
            pltpu.SemaphoreType.DMA((CH * NP,)),
            pltpu.SemaphoreType.DMA((CH * NP,)),
            pltpu.SemaphoreType.DMA((CH * NP,)),
            pltpu.SemaphoreType.DMA((CH * NP,)),
        ],
        compiler_params=pltpu.CompilerParams(collective_id=0),
    )(x16, Wq16, K_sh, V_sh, Wo16)


# device time: 15219 ns/iter; 1.7004x vs baseline; 1.1493x over previous
import jax
import jax.numpy as jnp
from jax import lax
from jax.experimental import pallas as pl
from jax.experimental.pallas import tpu as pltpu

N_DEV = 4
B, SQ, SKV, HQ_LOCAL, DH = 2, 256, 256, 4, 64
DMODEL = 512
WINDOW = 128
SCALE = 0.125
CH = 4
QR = SQ // N_DEV
CW = DMODEL // CH
NP = N_DEV - 1


def kernel(x, Wq, K_ext, V_ext, Wo):
    my = lax.axis_index("i")
    K_sh = lax.dynamic_slice_in_dim(K_ext, my * HQ_LOCAL, HQ_LOCAL, axis=2)
    V_sh = lax.dynamic_slice_in_dim(V_ext, my * HQ_LOCAL, HQ_LOCAL, axis=2)
    K_sh = jnp.transpose(K_sh, (0, 2, 1, 3)).astype(jnp.bfloat16)
    V_sh = jnp.transpose(V_sh, (0, 2, 1, 3)).astype(jnp.bfloat16)

    def body(x_ref, wq_ref, k_ref, v_ref, wo_ref, out_ref,
             ctx_ref, part_ref, fin_ref, scat_ref,
             send_a, recv_a, send_b, recv_b):
        me = lax.axis_index("i")

        barrier_sem = pltpu.get_barrier_semaphore()
        for d in range(1, N_DEV):
            pl.semaphore_signal(
                barrier_sem, inc=1,
                device_id=((me + d) % N_DEV,),
                device_id_type=pl.DeviceIdType.MESH,
            )

        qi = lax.broadcasted_iota(jnp.int32, (SQ, SKV), 0)
        ki = lax.broadcasted_iota(jnp.int32, (SQ, SKV), 1)
        mask = jnp.abs(qi - ki) <= WINDOW

        wq = wq_ref[:, :].astype(jnp.bfloat16)
        wo = wo_ref[:, :].astype(jnp.bfloat16)
        for b in range(B):
            q_b = jnp.dot(x_ref[b].astype(jnp.bfloat16), wq,
                          preferred_element_type=jnp.float32)
            q_b16 = q_b.astype(jnp.bfloat16)
            for h in range(HQ_LOCAL):
                q_bh = q_b16[:, h * DH:(h + 1) * DH]
                k_bh = k_ref[b, h]
                s = lax.dot_general(
                    q_bh, k_bh, (((1,), (1,)), ((), ())),
                    preferred_element_type=jnp.float32,
                ) * SCALE
                w = jnp.where(mask, jnp.exp(s), 0.0)
                w = w / jnp.sum(w, axis=1, keepdims=True)
                ctx_ref[b, :, h * DH:(h + 1) * DH] = jnp.dot(
                    w.astype(jnp.bfloat16), v_ref[b, h],
                    preferred_element_type=jnp.float32,
                ).astype(jnp.bfloat16)
        for b in range(B):
            p_b = jnp.dot(ctx_ref[b], wo, preferred_element_type=jnp.float32)
            out_ref[b] = p_b
            part_ref[b] = p_b.astype(jnp.bfloat16)

        my_rows = pl.ds(me * QR, QR)

        pl.semaphore_wait(barrier_sem, N_DEV - 1)

        sends = []
        for c in range(CH):
            cols = pl.ds(c * CW, CW)
            for d in range(1, N_DEV):
                t = (me + d) % N_DEV
                rdma = pltpu.make_async_remote_copy(
                    src_ref=part_ref.at[:, pl.ds(t * QR, QR), cols],
                    dst_ref=scat_ref.at[N_DEV - 1 - d, :, :, cols],
                    send_sem=send_a.at[c * NP + d - 1],
                    recv_sem=recv_a.at[c * NP + N_DEV - 1 - d],
                    device_id=(t,),
                    device_id_type=pl.DeviceIdType.MESH,
                )
                rdma.start()
                sends.append(rdma)

        for c in range(CH):
            cols = pl.ds(c * CW, CW)
            for r in range(NP):
                pltpu.make_async_remote_copy(
                    src_ref=scat_ref.at[r, :, :, cols],
                    dst_ref=scat_ref.at[r, :, :, cols],
                    send_sem=send_a.at[c * NP + r],
                    recv_sem=recv_a.at[c * NP + r],
                    device_id=(me,), device_id_type=pl.DeviceIdType.MESH,
                ).wait_recv()
            red = (
                out_ref[:, my_rows, cols]
                + scat_ref[0, :, :, cols].astype(jnp.float32)
                + scat_ref[1, :, :, cols].astype(jnp.float32)
                + scat_ref[2, :, :, cols].astype(jnp.float32)
            )
            out_ref[:, my_rows, cols] = red
            fin_ref[:, my_rows, cols] = red.astype(jnp.bfloat16)
            for d in range(1, N_DEV):
                t = (me + d) % N_DEV
                rdma = pltpu.make_async_remote_copy(
                    src_ref=fin_ref.at[:, my_rows, cols],
                    dst_ref=fin_ref.at[:, my_rows, cols],
                    send_sem=send_b.at[c * NP + d - 1],
                    recv_sem=recv_b.at[c * NP + N_DEV - 1 - d],
                    device_id=(t,),
                    device_id_type=pl.DeviceIdType.MESH,
                )
                rdma.start()
                sends.append(rdma)

        for c in range(CH):
            cols = pl.ds(c * CW, CW)
            for r in range(NP):
                s = (me + 1 + r) % N_DEV
                pltpu.make_async_remote_copy(
                    src_ref=fin_ref.at[:, pl.ds(s * QR, QR), cols],
                    dst_ref=fin_ref.at[:, pl.ds(s * QR, QR), cols],
                    send_sem=send_b.at[c * NP + r],
                    recv_sem=recv_b.at[c * NP + r],
                    device_id=(me,), device_id_type=pl.DeviceIdType.MESH,
                ).wait_recv()
        for r in range(NP):
            s_rows = pl.ds(((me + 1 + r) % N_DEV) * QR, QR)
            out_ref[:, s_rows, :] = fin_ref[:, s_rows, :].astype(jnp.float32)
        for rdma in sends:
            rdma.wait_send()

    return pl.pallas_call(
        body,
        out_shape=jax.ShapeDtypeStruct((B, SQ, DMODEL), jnp.float32),
        in_specs=[pl.BlockSpec(memory_space=pltpu.VMEM)] * 5,
        out_specs=pl.BlockSpec(memory_space=pltpu.VMEM),
        scratch_shapes=[
            pltpu.VMEM((B, SQ, HQ_LOCAL * DH), jnp.bfloat16),
            pltpu.VMEM((B, SQ, DMODEL), jnp.bfloat16),
            pltpu.VMEM((B, SQ, DMODEL), jnp.bfloat16),
            pltpu.VMEM((NP, B, QR, DMODEL), jnp.bfloat16),
            pltpu.SemaphoreType.DMA((CH * NP,)),
            pltpu.SemaphoreType.DMA((CH * NP,)),
            pltpu.SemaphoreType.DMA((CH * NP,)),
            pltpu.SemaphoreType.DMA((CH * NP,)),
        ],
        compiler_params=pltpu.CompilerParams(collective_id=0),
    )(x, Wq, K_sh, V_sh, Wo)
